# baseline (speedup 1.0000x reference)
import functools

import jax
import jax.numpy as jnp
from jax import lax
from jax.experimental import pallas as pl
from jax.experimental.pallas import tpu as pltpu
from jax.experimental.pallas import tpu_sc as plsc

DIM = 32
M = 8192
K = 8192
TM = 512
NT = M // TM

_NC = 2
_NS = 16
_NW = _NC * _NS
_BPW = M // _NW


def _dist_body(x_ref, ew_ref, w_ref, b_ref,
               d_ref, idx_ref, diff_ref, cbp_ref, cb_scr, c2_scr):
    i = pl.program_id(0)

    @pl.when(i == 0)
    def _project():
        proj = lax.dot_general(
            ew_ref[...], w_ref[...], (((1,), (1,)), ((), ())),
            precision=lax.Precision.DEFAULT,
            preferred_element_type=jnp.float32) + b_ref[...]
        cb_scr[...] = proj
        c2_scr[...] = jnp.sum(proj * proj, axis=1).reshape(1, K)
        cbp_ref[...] = jnp.concatenate(
            [proj, jnp.zeros((K, 128 - DIM), jnp.float32)], axis=1)
        diff_ref[...] = jnp.zeros((1, 1), jnp.float32)

    x = x_ref[...]
    x2 = jnp.sum(x * x, axis=1, keepdims=True)
    xc2 = lax.dot_general(
        x * (-2.0), cb_scr[...], (((1,), (1,)), ((), ())),
        precision=lax.Precision.DEFAULT,
        preferred_element_type=jnp.float32)
    d = (x2 + c2_scr[...]) + xc2
    d_ref[...] = d
    dmin = jnp.min(d[:, :128], axis=1, keepdims=True)
    idx_ref[0, 0, :] = dmin.reshape(TM).astype(jnp.int32)

    diff_ref[...] += jnp.sum(dmin).reshape(1, 1)

    @pl.when(i == NT - 1)
    def _finalize():
        s = diff_ref[...] / (M * DIM)
        diff_ref[...] = s + 0.25 * s


_HBPW = _BPW // 2


def _sc_gather_body(table_hbm, idx_hbm, out_hbm,
                    idx_v0, idx_v1, rows_v0, rows_v1,
                    sem0, sem1, osem0, osem1):
    wid = lax.axis_index("s") * _NC + lax.axis_index("c")
    base = wid * _BPW
    pltpu.sync_copy(idx_hbm.at[pl.ds(base, _HBPW)], idx_v0)
    g0 = pltpu.async_copy(table_hbm.at[idx_v0], rows_v0, sem0)
    pltpu.sync_copy(idx_hbm.at[pl.ds(base + _HBPW, _HBPW)], idx_v1)
    g1 = pltpu.async_copy(table_hbm.at[idx_v1], rows_v1, sem1)
    g0.wait()
    s0 = pltpu.async_copy(rows_v0, out_hbm.at[pl.ds(base, _HBPW)], osem0)
    g1.wait()
    s1 = pltpu.async_copy(
        rows_v1, out_hbm.at[pl.ds(base + _HBPW, _HBPW)], osem1)
    s0.wait()
    s1.wait()


def _make_sc_gather():
    return functools.partial(
        pl.kernel,
        mesh=plsc.VectorSubcoreMesh(core_axis_name="c", subcore_axis_name="s"),
        out_type=jax.ShapeDtypeStruct((M, 128), jnp.float32),
        scratch_types=[
            pltpu.VMEM((_HBPW,), jnp.int32),
            pltpu.VMEM((_HBPW,), jnp.int32),
            pltpu.VMEM((_HBPW, 128), jnp.float32),
            pltpu.VMEM((_HBPW, 128), jnp.float32),
            pltpu.SemaphoreType.DMA,
            pltpu.SemaphoreType.DMA,
            pltpu.SemaphoreType.DMA,
            pltpu.SemaphoreType.DMA,
        ],
    )(_sc_gather_body)


def kernel(input, is_look_back, embed_weight, proj_w, proj_b):
    flatten = input.reshape(-1, DIM)

    d, idx3, diff11, cb_pad = pl.pallas_call(
        _dist_body,
        grid=(NT,),
        in_specs=[
            pl.BlockSpec((TM, DIM), lambda i: (i, 0)),
            pl.BlockSpec((K, DIM), lambda i: (0, 0)),
            pl.BlockSpec((DIM, DIM), lambda i: (0, 0)),
            pl.BlockSpec((1, DIM), lambda i: (0, 0)),
        ],
        out_specs=[
            pl.BlockSpec((TM, K), lambda i: (i, 0)),
            pl.BlockSpec((1, 1, TM), lambda i: (i, 0, 0)),
            pl.BlockSpec((1, 1), lambda i: (0, 0)),
            pl.BlockSpec((K, 128), lambda i: (0, 0)),
        ],
        out_shape=[
            jax.ShapeDtypeStruct((M, K), jnp.float32),
            jax.ShapeDtypeStruct((NT, 1, TM), jnp.int32),
            jax.ShapeDtypeStruct((1, 1), jnp.float32),
            jax.ShapeDtypeStruct((K, 128), jnp.float32),
        ],
        scratch_shapes=[
            pltpu.VMEM((K, DIM), jnp.float32),
            pltpu.VMEM((1, K), jnp.float32),
        ],
    )(flatten, embed_weight, proj_w, proj_b.reshape(1, DIM))

    idx = idx3.reshape(M)
    z_quantize = _make_sc_gather()(cb_pad, idx)[:, :DIM].reshape(input.shape)
    diff = diff11.reshape(())
    embed_ind = idx.reshape(input.shape[:-1])
    return (z_quantize, diff, embed_ind, d)


# speedup vs baseline: 1.0760x; 1.0760x over previous
import functools

import jax
import jax.numpy as jnp
from jax import lax
from jax.experimental import pallas as pl
from jax.experimental.pallas import tpu as pltpu
from jax.experimental.pallas import tpu_sc as plsc

DIM = 32
M = 8192
K = 8192
TM = 512
NT = M // TM

_NC = 2
_NS = 16
_NW = _NC * _NS
_BPW = M // _NW


def _dist_body(x_ref, ew_ref, w_ref, b_ref,
               d_ref, idx_ref, diff_ref, cbp_ref, cb_scr, c2_scr):
    i = pl.program_id(0)

    @pl.when(i == 0)
    def _project():
        proj = lax.dot_general(
            ew_ref[...], w_ref[...], (((1,), (1,)), ((), ())),
            precision=lax.Precision.DEFAULT,
            preferred_element_type=jnp.float32) + b_ref[...]
        cb_scr[...] = proj
        c2_scr[...] = jnp.sum(proj * proj, axis=1).reshape(1, K)
        cbp_ref[...] = jnp.concatenate(
            [proj, jnp.zeros((K, 128 - DIM), jnp.float32)], axis=1)
        diff_ref[...] = jnp.zeros((1, 1), jnp.float32)

    x = x_ref[...]
    x2 = jnp.sum(x * x, axis=1, keepdims=True)
    xc2 = lax.dot_general(
        x * (-2.0), cb_scr[...], (((1,), (1,)), ((), ())),
        precision=lax.Precision.DEFAULT,
        preferred_element_type=jnp.float32)
    d = (x2 + c2_scr[...]) + xc2
    d_ref[...] = d
    dmin = jnp.min(d, axis=1, keepdims=True)
    iota = lax.broadcasted_iota(jnp.int32, d.shape, 1).astype(jnp.float32)
    idxf = jnp.min(jnp.where(d == dmin, iota, float(K)), axis=1)
    idx_ref[0, 0, :] = idxf.astype(jnp.int32)

    diff_ref[...] += jnp.sum(dmin).reshape(1, 1)

    @pl.when(i == NT - 1)
    def _finalize():
        s = diff_ref[...] / (M * DIM)
        diff_ref[...] = s + 0.25 * s


_HBPW = _BPW // 2


def _sc_gather_body(table_hbm, idx_hbm, out_hbm,
                    idx_v0, idx_v1, rows_v0, rows_v1,
                    sem0, sem1, osem0, osem1):
    wid = lax.axis_index("s") * _NC + lax.axis_index("c")
    base = wid * _BPW
    pltpu.sync_copy(idx_hbm.at[pl.ds(base, _HBPW)], idx_v0)
    g0 = pltpu.async_copy(table_hbm.at[idx_v0], rows_v0, sem0)
    pltpu.sync_copy(idx_hbm.at[pl.ds(base + _HBPW, _HBPW)], idx_v1)
    g1 = pltpu.async_copy(table_hbm.at[idx_v1], rows_v1, sem1)
    g0.wait()
    s0 = pltpu.async_copy(rows_v0, out_hbm.at[pl.ds(base, _HBPW)], osem0)
    g1.wait()
    s1 = pltpu.async_copy(
        rows_v1, out_hbm.at[pl.ds(base + _HBPW, _HBPW)], osem1)
    s0.wait()
    s1.wait()


def _make_sc_gather():
    return functools.partial(
        pl.kernel,
        mesh=plsc.VectorSubcoreMesh(core_axis_name="c", subcore_axis_name="s"),
        out_type=jax.ShapeDtypeStruct((M, 128), jnp.float32),
        scratch_types=[
            pltpu.VMEM((_HBPW,), jnp.int32),
            pltpu.VMEM((_HBPW,), jnp.int32),
            pltpu.VMEM((_HBPW, 128), jnp.float32),
            pltpu.VMEM((_HBPW, 128), jnp.float32),
            pltpu.SemaphoreType.DMA,
            pltpu.SemaphoreType.DMA,
            pltpu.SemaphoreType.DMA,
            pltpu.SemaphoreType.DMA,
        ],
    )(_sc_gather_body)


def kernel(input, is_look_back, embed_weight, proj_w, proj_b):
    flatten = input.reshape(-1, DIM)

    d, idx3, diff11, cb_pad = pl.pallas_call(
        _dist_body,
        grid=(NT,),
        in_specs=[
            pl.BlockSpec((TM, DIM), lambda i: (i, 0)),
            pl.BlockSpec((K, DIM), lambda i: (0, 0)),
            pl.BlockSpec((DIM, DIM), lambda i: (0, 0)),
            pl.BlockSpec((1, DIM), lambda i: (0, 0)),
        ],
        out_specs=[
            pl.BlockSpec((TM, K), lambda i: (i, 0)),
            pl.BlockSpec((1, 1, TM), lambda i: (i, 0, 0)),
            pl.BlockSpec((1, 1), lambda i: (0, 0)),
            pl.BlockSpec((K, 128), lambda i: (0, 0)),
        ],
        out_shape=[
            jax.ShapeDtypeStruct((M, K), jnp.float32),
            jax.ShapeDtypeStruct((NT, 1, TM), jnp.int32),
            jax.ShapeDtypeStruct((1, 1), jnp.float32),
            jax.ShapeDtypeStruct((K, 128), jnp.float32),
        ],
        scratch_shapes=[
            pltpu.VMEM((K, DIM), jnp.float32),
            pltpu.VMEM((1, K), jnp.float32),
        ],
    )(flatten, embed_weight, proj_w, proj_b.reshape(1, DIM))

    idx = idx3.reshape(M)
    z_quantize = _make_sc_gather()(cb_pad, idx)[:, :DIM].reshape(input.shape)
    diff = diff11.reshape(())
    embed_ind = idx.reshape(input.shape[:-1])
    return (z_quantize, diff, embed_ind, d)
